# Initial kernel scaffold; baseline (speedup 1.0000x reference)
import jax
import jax.numpy as jnp
from jax.experimental import pallas as pl


def kernel(input_ids, weight):
    raise NotImplementedError("write your pallas kernel here")


# speedup vs baseline: 1.5648x; 1.5648x over previous
import functools

import jax
import jax.numpy as jnp
from jax import lax
from jax.experimental import pallas as pl
from jax.experimental.pallas import tpu as pltpu
from jax.experimental.pallas import tpu_sc as plsc

_D = 768
_NC = 2
_NS = 16
_NW = _NC * _NS
_CHUNK = 64
_NCHUNK = 100


def _emb_body(idx_hbm, table_hbm, out_hbm, idx_v, rows0, rows1,
              gsem0, gsem1, osem0, osem1):
    wid = lax.axis_index("s") * _NC + lax.axis_index("c")
    pltpu.sync_copy(idx_hbm.at[wid], idx_v)

    rows = (rows0, rows1)
    gsem = (gsem0, gsem1)
    osem = (osem0, osem1)

    def start_gather(g, b):
        pltpu.make_async_copy(
            table_hbm.at[idx_v.at[g]], rows[b], gsem[b]).start()

    def wait_gather(b):
        pltpu.make_async_copy(
            table_hbm.at[idx_v.at[0]], rows[b], gsem[b]).wait()

    def start_out(g, b):
        pltpu.make_async_copy(rows[b], out_hbm.at[wid, g], osem[b]).start()

    def wait_out(b):
        pltpu.make_async_copy(rows[b], out_hbm.at[wid, 0], osem[b]).wait()

    start_gather(0, 0)

    def ring(i, carry):
        g0 = 2 * i
        g1 = g0 + 1
        @pl.when(i > 0)
        def _():
            wait_out(1)
        start_gather(g1, 1)
        wait_gather(0)
        start_out(g0, 0)
        wait_out(0)

        @pl.when(g1 + 1 < _NCHUNK)
        def _():
            start_gather(g1 + 1, 0)
        wait_gather(1)
        start_out(g1, 1)
        return carry

    lax.fori_loop(0, _NCHUNK // 2, ring, 0, unroll=False)
    wait_out(1)


@functools.partial(jax.jit, static_argnames=())
def _emb_call(idx, weight):
    mesh = plsc.VectorSubcoreMesh(core_axis_name="c", subcore_axis_name="s")
    return pl.kernel(
        _emb_body,
        out_type=jax.ShapeDtypeStruct((_NW, _NCHUNK, _CHUNK, _D), jnp.float32),
        mesh=mesh,
        scratch_types=[
            pltpu.VMEM((_NCHUNK, _CHUNK), jnp.int32),
            pltpu.VMEM((_CHUNK, _D), jnp.float32),
            pltpu.VMEM((_CHUNK, _D), jnp.float32),
            pltpu.SemaphoreType.DMA,
            pltpu.SemaphoreType.DMA,
            pltpu.SemaphoreType.DMA,
            pltpu.SemaphoreType.DMA,
        ],
    )(idx, weight)


def kernel(input_ids, weight):
    idx = input_ids.reshape(_NW, _NCHUNK, _CHUNK).astype(jnp.int32)
    out = _emb_call(idx, weight)
    return out.reshape(input_ids.shape + (_D,))


# speedup vs baseline: 1.5685x; 1.0024x over previous
import functools

import jax
import jax.numpy as jnp
from jax import lax
from jax.experimental import pallas as pl
from jax.experimental.pallas import tpu as pltpu
from jax.experimental.pallas import tpu_sc as plsc

_D = 768
_NC = 2
_NS = 16
_NW = _NC * _NS
_CHUNK = 32
_NCHUNK = 200
_NBUF = 4


def _emb_body(idx_hbm, table_hbm, out_hbm, idx_v,
              rows0, rows1, rows2, rows3,
              gsem0, gsem1, gsem2, gsem3,
              osem0, osem1, osem2, osem3):
    wid = lax.axis_index("s") * _NC + lax.axis_index("c")
    pltpu.sync_copy(idx_hbm.at[wid], idx_v)

    rows = (rows0, rows1, rows2, rows3)
    gsem = (gsem0, gsem1, gsem2, gsem3)
    osem = (osem0, osem1, osem2, osem3)

    def start_gather(g, b):
        pltpu.make_async_copy(
            table_hbm.at[idx_v.at[g]], rows[b], gsem[b]).start()

    def wait_gather(b):
        pltpu.make_async_copy(
            table_hbm.at[idx_v.at[0]], rows[b], gsem[b]).wait()

    def start_out(g, b):
        pltpu.make_async_copy(rows[b], out_hbm.at[wid, g], osem[b]).start()

    def wait_out(b):
        pltpu.make_async_copy(rows[b], out_hbm.at[wid, 0], osem[b]).wait()

    for b in range(_NBUF):
        start_gather(b, b)

    def ring(i, carry):
        for b in range(_NBUF):
            g = i * _NBUF + b
            bprev = (b - 1) % _NBUF
            gprev = g - 1

            @pl.when(gprev >= 0)
            def _():
                wait_out(bprev)

            @pl.when((gprev >= 0) & (gprev + _NBUF < _NCHUNK))
            def _():
                start_gather(gprev + _NBUF, bprev)

            wait_gather(b)
            start_out(g, b)
        return carry

    lax.fori_loop(0, _NCHUNK // _NBUF, ring, 0, unroll=False)
    wait_out(_NBUF - 1)


@functools.partial(jax.jit, static_argnames=())
def _emb_call(idx, weight):
    mesh = plsc.VectorSubcoreMesh(core_axis_name="c", subcore_axis_name="s")
    return pl.kernel(
        _emb_body,
        out_type=jax.ShapeDtypeStruct((_NW, _NCHUNK, _CHUNK, _D), jnp.float32),
        mesh=mesh,
        scratch_types=(
            [pltpu.VMEM((_NCHUNK, _CHUNK), jnp.int32)]
            + [pltpu.VMEM((_CHUNK, _D), jnp.float32)] * _NBUF
            + [pltpu.SemaphoreType.DMA] * (2 * _NBUF)
        ),
    )(idx, weight)


def kernel(input_ids, weight):
    idx = input_ids.reshape(_NW, _NCHUNK, _CHUNK).astype(jnp.int32)
    out = _emb_call(idx, weight)
    return out.reshape(input_ids.shape + (_D,))


# speedup vs baseline: 2.5276x; 1.6115x over previous
import functools

import jax
import jax.numpy as jnp
from jax import lax
from jax.experimental import pallas as pl
from jax.experimental.pallas import tpu as pltpu
from jax.experimental.pallas import tpu_sc as plsc

_D = 768
_NC = 2
_NS = 16
_NW = _NC * _NS
_CHUNK = 50
_NCHUNK = 128
_NBUF = 2


def _emb_body(idx_hbm, table_hbm, out_hbm, idx_v,
              rows0, rows1,
              gsem0, gsem1,
              osem0, osem1):
    wid = lax.axis_index("s") * _NC + lax.axis_index("c")
    base = wid * _NCHUNK
    pltpu.sync_copy(idx_hbm.at[wid], idx_v)

    rows = (rows0, rows1)
    gsem = (gsem0, gsem1)
    osem = (osem0, osem1)

    def start_gather(g, b):
        pltpu.make_async_copy(
            table_hbm.at[idx_v.at[g]], rows[b], gsem[b]).start()

    def wait_gather(b):
        pltpu.make_async_copy(
            table_hbm.at[idx_v.at[0]], rows[b], gsem[b]).wait()

    def start_out(g, b):
        pltpu.make_async_copy(rows[b], out_hbm.at[base + g], osem[b]).start()

    def wait_out(b):
        pltpu.make_async_copy(rows[b], out_hbm.at[base], osem[b]).wait()

    for b in range(_NBUF):
        start_gather(b, b)

    def ring(i, carry):
        for b in range(_NBUF):
            g = i * _NBUF + b
            bprev = (b - 1) % _NBUF
            gprev = g - 1

            @pl.when(gprev >= 0)
            def _():
                wait_out(bprev)

            @pl.when((gprev >= 0) & (gprev + _NBUF < _NCHUNK))
            def _():
                start_gather(gprev + _NBUF, bprev)

            wait_gather(b)
            start_out(g, b)
        return carry

    lax.fori_loop(0, _NCHUNK // _NBUF, ring, 0, unroll=False)
    wait_out(_NBUF - 1)


@jax.jit
def _emb_call(idx, weight):
    mesh = plsc.VectorSubcoreMesh(core_axis_name="c", subcore_axis_name="s")
    return pl.kernel(
        _emb_body,
        out_type=jax.ShapeDtypeStruct((_NW * _NCHUNK, _CHUNK, _D),
                                      jnp.float32),
        mesh=mesh,
        scratch_types=(
            [pltpu.VMEM((_NCHUNK, _CHUNK), jnp.int32)]
            + [pltpu.VMEM((_CHUNK, _D), jnp.float32)] * _NBUF
            + [pltpu.SemaphoreType.DMA] * (2 * _NBUF)
        ),
    )(idx, weight)


def kernel(input_ids, weight):
    idx = input_ids.reshape(_NW, _NCHUNK, _CHUNK).astype(jnp.int32)
    return _emb_call(idx, weight)
